# baseline (speedup 1.0000x reference)
import functools

import jax
import jax.numpy as jnp
from jax import lax
from jax.experimental import pallas as pl
from jax.experimental.pallas import tpu as pltpu
from jax.experimental.pallas import tpu_sc as plsc

B = 1024
L = 50
VOCAB = 100000
EMBDIM = 64
HDIM = 128
MAXLEN = 51
NUMSTEPS = 10

N = B * L

_NC = 2
_NS = 16
_NW = _NC * _NS
_ROWS_PER_W = N // _NW
_CHUNK = 80
_NCHUNK = _ROWS_PER_W // _CHUNK


def _sc_gather(emb128, idx3):
    mesh = plsc.VectorSubcoreMesh(core_axis_name="c", subcore_axis_name="s")

    @functools.partial(
        pl.kernel,
        mesh=mesh,
        out_type=jax.ShapeDtypeStruct((N, HDIM), jnp.float32),
        scratch_types=[
            pltpu.VMEM((_NCHUNK, _CHUNK), jnp.int32),
            pltpu.VMEM((_CHUNK, HDIM), jnp.float32),
            pltpu.SemaphoreType.DMA,
        ],
    )
    def gather_kernel(emb_hbm, idx_hbm, out_hbm, idx_v, row_v, sem):
        wid = lax.axis_index("s") * _NC + lax.axis_index("c")
        base = wid * _ROWS_PER_W
        pltpu.sync_copy(idx_hbm.at[wid], idx_v)
        for j in range(_NCHUNK):
            pltpu.async_copy(emb_hbm.at[idx_v.at[j]], row_v, sem).wait()
            pltpu.sync_copy(row_v, out_hbm.at[pl.ds(base + j * _CHUNK, _CHUNK)])

    return gather_kernel(emb128, idx3)


_BBLK = 256


def _prop_body(e_ref, p_ref, w12v_ref, wzr2_ref, wuh2_ref,
               b12_ref, bzrh_ref, out_ref):
    m = L * _BBLK
    pos = jnp.broadcast_to(p_ref[...][:, None, :], (L, _BBLK, HDIM))
    h = (e_ref[...] + pos).reshape(m, HDIM)
    w12v = w12v_ref[...].astype(jnp.bfloat16)
    wzr2 = wzr2_ref[...].astype(jnp.bfloat16)
    wuh2 = wuh2_ref[...].astype(jnp.bfloat16)
    b12 = b12_ref[...]
    bzrh = bzrh_ref[...]
    row = lax.broadcasted_iota(jnp.int32, (m, 1), 0)
    maskf = (row >= _BBLK).astype(jnp.float32)
    maskb = (row < m - _BBLK).astype(jnp.float32)
    bmsg = maskf * b12[:, :HDIM] + maskb * b12[:, HDIM:]
    zpad = jnp.zeros((_BBLK, HDIM), jnp.bfloat16)
    for _ in range(NUMSTEPS):
        hb = h.astype(jnp.bfloat16)
        hshift = jnp.concatenate(
            [jnp.concatenate([zpad, hb[: m - _BBLK]], axis=0),
             jnp.concatenate([hb[_BBLK:], zpad], axis=0)], axis=1)
        agg = jnp.dot(hshift, w12v, preferred_element_type=jnp.float32) + bmsg
        ab = agg.astype(jnp.bfloat16)
        zr = jnp.dot(jnp.concatenate([hb, ab], axis=1), wzr2,
                     preferred_element_type=jnp.float32) + bzrh[:, :2 * HDIM]
        z = jax.nn.sigmoid(zr[:, :HDIM])
        r = jax.nn.sigmoid(zr[:, HDIM:])
        hh = jnp.tanh(
            jnp.dot(jnp.concatenate([ab, (r * h).astype(jnp.bfloat16)], axis=1),
                    wuh2, preferred_element_type=jnp.float32)
            + bzrh[:, 2 * HDIM:])
        h = (1.0 - z) * h + z * hh
    out_ref[...] = h[(L - 1) * _BBLK:, :]


def _propagate(embs3, pos2d, w12v, wzr2, wuh2, b12, bzrh):
    grid = B // _BBLK
    return pl.pallas_call(
        _prop_body,
        grid=(grid,),
        in_specs=[
            pl.BlockSpec((L, _BBLK, HDIM), lambda i: (0, i, 0)),
            pl.BlockSpec((L, HDIM), lambda i: (0, 0)),
            pl.BlockSpec((2 * HDIM, HDIM), lambda i: (0, 0)),
            pl.BlockSpec((2 * HDIM, 2 * HDIM), lambda i: (0, 0)),
            pl.BlockSpec((2 * HDIM, HDIM), lambda i: (0, 0)),
            pl.BlockSpec((1, 2 * HDIM), lambda i: (0, 0)),
            pl.BlockSpec((1, 3 * HDIM), lambda i: (0, 0)),
        ],
        out_specs=pl.BlockSpec((_BBLK, HDIM), lambda i: (i, 0)),
        out_shape=jax.ShapeDtypeStruct((B, HDIM), jnp.float32),
    )(embs3, pos2d, w12v, wzr2, wuh2, b12, bzrh)


_VBLK = 1024


def _proj_body(lo_ref, w_ref, b_ref, o_ref):
    o_ref[...] = (
        lax.dot_general(lo_ref[...], w_ref[...], (((1,), (1,)), ((), ())),
                        preferred_element_type=jnp.float32)
        + b_ref[...]
    )


def _project(lastout, wout, bout2):
    grid = pl.cdiv(VOCAB, _VBLK)
    return pl.pallas_call(
        _proj_body,
        grid=(grid,),
        in_specs=[
            pl.BlockSpec((B, HDIM), lambda i: (0, 0)),
            pl.BlockSpec((_VBLK, HDIM), lambda i: (i, 0)),
            pl.BlockSpec((1, _VBLK), lambda i: (0, i)),
        ],
        out_specs=pl.BlockSpec((B, _VBLK), lambda i: (0, i)),
        out_shape=jax.ShapeDtypeStruct((B, VOCAB), jnp.float32),
    )(lastout, wout, bout2)


def kernel(x, emb, posemb, W_msg, b_msg, Wz, Uz, bz, Wr, Ur, br, Wh, Uh, bh,
           Wout, bout):
    idx3 = x.T.reshape(_NW, _NCHUNK, _CHUNK)
    emb128 = jnp.concatenate([emb, jnp.zeros_like(emb)], axis=1)
    embs3 = _sc_gather(emb128, idx3).reshape(L, B, HDIM)
    pos2d = jnp.concatenate(
        [jnp.zeros((L, EMBDIM), jnp.float32), posemb[1:L + 1]], axis=1)

    w12v = jnp.concatenate([W_msg[1], W_msg[2]], axis=0)
    wzr2 = jnp.concatenate(
        [jnp.concatenate([Uz, Ur], axis=1),
         jnp.concatenate([Wz, Wr], axis=1)], axis=0)
    wuh2 = jnp.concatenate([Wh, Uh], axis=0)
    b12 = jnp.concatenate([b_msg[1], b_msg[2]])[None, :]
    bzrh = jnp.concatenate([bz, br, bh])[None, :]

    lastout = _propagate(embs3, pos2d, w12v, wzr2, wuh2, b12, bzrh)
    return _project(lastout, Wout, bout[None, :])


# speedup vs baseline: 1.0276x; 1.0276x over previous
import functools

import jax
import jax.numpy as jnp
from jax import lax
from jax.experimental import pallas as pl
from jax.experimental.pallas import tpu as pltpu
from jax.experimental.pallas import tpu_sc as plsc

B = 1024
L = 50
VOCAB = 100000
EMBDIM = 64
HDIM = 128
MAXLEN = 51
NUMSTEPS = 10

N = B * L

_NC = 2
_NS = 16
_NW = _NC * _NS
_ROWS_PER_W = N // _NW
_CHUNK = 80
_NCHUNK = _ROWS_PER_W // _CHUNK


def _sc_gather(emb128, idx3):
    mesh = plsc.VectorSubcoreMesh(core_axis_name="c", subcore_axis_name="s")

    @functools.partial(
        pl.kernel,
        mesh=mesh,
        out_type=jax.ShapeDtypeStruct((N, HDIM), jnp.float32),
        scratch_types=[
            pltpu.VMEM((_NCHUNK, _CHUNK), jnp.int32),
            pltpu.VMEM((_CHUNK, HDIM), jnp.float32),
            pltpu.SemaphoreType.DMA,
        ],
    )
    def gather_kernel(emb_hbm, idx_hbm, out_hbm, idx_v, row_v, sem):
        wid = lax.axis_index("s") * _NC + lax.axis_index("c")
        base = wid * _ROWS_PER_W
        pltpu.sync_copy(idx_hbm.at[wid], idx_v)
        for j in range(_NCHUNK):
            pltpu.async_copy(emb_hbm.at[idx_v.at[j]], row_v, sem).wait()
            pltpu.sync_copy(row_v, out_hbm.at[pl.ds(base + j * _CHUNK, _CHUNK)])

    return gather_kernel(emb128, idx3)


_BBLK = 256


def _prop_body(e_ref, p_ref, w12v_ref, wzr2_ref, wuh2_ref,
               b12_ref, bzrh_ref, out_ref):
    m = L * _BBLK
    pos = jnp.broadcast_to(p_ref[...][:, None, :], (L, _BBLK, HDIM))
    h = (e_ref[...] + pos).reshape(m, HDIM)
    w12v = w12v_ref[...].astype(jnp.bfloat16)
    wzr2 = wzr2_ref[...].astype(jnp.bfloat16)
    wuh2 = wuh2_ref[...].astype(jnp.bfloat16)
    b12 = b12_ref[...]
    bzrh = bzrh_ref[...]
    row = lax.broadcasted_iota(jnp.int32, (m, 1), 0)
    maskf = (row >= _BBLK).astype(jnp.float32)
    maskb = (row < m - _BBLK).astype(jnp.float32)
    bmsg = maskf * b12[:, :HDIM] + maskb * b12[:, HDIM:]
    zpad = jnp.zeros((_BBLK, HDIM), jnp.bfloat16)
    for _ in range(NUMSTEPS):
        hb = h.astype(jnp.bfloat16)
        hshift = jnp.concatenate(
            [jnp.concatenate([zpad, hb[: m - _BBLK]], axis=0),
             jnp.concatenate([hb[_BBLK:], zpad], axis=0)], axis=1)
        agg = jnp.dot(hshift, w12v, preferred_element_type=jnp.float32) + bmsg
        ab = agg.astype(jnp.bfloat16)
        zr = jnp.dot(jnp.concatenate([hb, ab], axis=1), wzr2,
                     preferred_element_type=jnp.float32) + bzrh[:, :2 * HDIM]
        z = jax.nn.sigmoid(zr[:, :HDIM])
        r = jax.nn.sigmoid(zr[:, HDIM:])
        hh = jnp.tanh(
            jnp.dot(jnp.concatenate([ab, (r * h).astype(jnp.bfloat16)], axis=1),
                    wuh2, preferred_element_type=jnp.float32)
            + bzrh[:, 2 * HDIM:])
        h = (1.0 - z) * h + z * hh
    out_ref[...] = h[(L - 1) * _BBLK:, :]


def _propagate(embs3, pos2d, w12v, wzr2, wuh2, b12, bzrh):
    grid = B // _BBLK
    return pl.pallas_call(
        _prop_body,
        grid=(grid,),
        in_specs=[
            pl.BlockSpec((L, _BBLK, HDIM), lambda i: (0, i, 0)),
            pl.BlockSpec((L, HDIM), lambda i: (0, 0)),
            pl.BlockSpec((2 * HDIM, HDIM), lambda i: (0, 0)),
            pl.BlockSpec((2 * HDIM, 2 * HDIM), lambda i: (0, 0)),
            pl.BlockSpec((2 * HDIM, HDIM), lambda i: (0, 0)),
            pl.BlockSpec((1, 2 * HDIM), lambda i: (0, 0)),
            pl.BlockSpec((1, 3 * HDIM), lambda i: (0, 0)),
        ],
        out_specs=pl.BlockSpec((_BBLK, HDIM), lambda i: (i, 0)),
        out_shape=jax.ShapeDtypeStruct((B, HDIM), jnp.float32),
    )(embs3, pos2d, w12v, wzr2, wuh2, b12, bzrh)


_VBLK = 4096


def _proj_body(lo_ref, w_ref, b_ref, o_ref):
    o_ref[...] = (
        lax.dot_general(lo_ref[...], w_ref[...], (((1,), (1,)), ((), ())),
                        preferred_element_type=jnp.float32)
        + b_ref[...]
    )


def _project(lastout, wout, bout2):
    grid = pl.cdiv(VOCAB, _VBLK)
    return pl.pallas_call(
        _proj_body,
        grid=(grid,),
        in_specs=[
            pl.BlockSpec((B, HDIM), lambda i: (0, 0)),
            pl.BlockSpec((_VBLK, HDIM), lambda i: (i, 0)),
            pl.BlockSpec((1, _VBLK), lambda i: (0, i)),
        ],
        out_specs=pl.BlockSpec((B, _VBLK), lambda i: (0, i)),
        out_shape=jax.ShapeDtypeStruct((B, VOCAB), jnp.float32),
        compiler_params=pltpu.CompilerParams(
            dimension_semantics=("parallel",)),
    )(lastout, wout, bout2)


def kernel(x, emb, posemb, W_msg, b_msg, Wz, Uz, bz, Wr, Ur, br, Wh, Uh, bh,
           Wout, bout):
    idx3 = x.T.reshape(_NW, _NCHUNK, _CHUNK)
    emb128 = jnp.concatenate([emb, jnp.zeros_like(emb)], axis=1)
    embs3 = _sc_gather(emb128, idx3).reshape(L, B, HDIM)
    pos2d = jnp.concatenate(
        [jnp.zeros((L, EMBDIM), jnp.float32), posemb[1:L + 1]], axis=1)

    w12v = jnp.concatenate([W_msg[1], W_msg[2]], axis=0)
    wzr2 = jnp.concatenate(
        [jnp.concatenate([Uz, Ur], axis=1),
         jnp.concatenate([Wz, Wr], axis=1)], axis=0)
    wuh2 = jnp.concatenate([Wh, Uh], axis=0)
    b12 = jnp.concatenate([b_msg[1], b_msg[2]])[None, :]
    bzrh = jnp.concatenate([bz, br, bh])[None, :]

    lastout = _propagate(embs3, pos2d, w12v, wzr2, wuh2, b12, bzrh)
    return _project(lastout, Wout, bout[None, :])
